# baseline (speedup 1.0000x reference)
import jax
import jax.numpy as jnp
from jax import lax
from jax.experimental import pallas as pl

E = 16
TOP_K = 2
CONF_THRESH = 0.7
TILE = 2048


def _router_body(x_ref, w_ref, wts_ref, idx_ref):
    logits = jnp.dot(x_ref[...], w_ref[...], preferred_element_type=jnp.float32)
    lt = logits.T
    m = jnp.max(lt, axis=0, keepdims=True)
    e = jnp.exp(lt - m)
    z = jnp.sum(e, axis=0, keepdims=True)
    probs = e / z

    iota = lax.broadcasted_iota(jnp.int32, probs.shape, 0)
    big = jnp.int32(E)
    w1 = jnp.max(probs, axis=0, keepdims=True)
    i1 = jnp.min(jnp.where(probs == w1, iota, big), axis=0, keepdims=True)
    masked = jnp.where(iota == i1, jnp.float32(-1.0), probs)
    w2 = jnp.max(masked, axis=0, keepdims=True)
    i2 = jnp.min(jnp.where(masked == w2, iota, big), axis=0, keepdims=True)

    keep = w1 >= CONF_THRESH
    i1 = jnp.where(keep, i1, -1)
    i2 = jnp.where(keep, i2, -1)

    wts_ref[...] = jnp.concatenate([w1, w2], axis=0)
    idx_ref[...] = jnp.concatenate([i1, i2], axis=0)


def kernel(x, W_g):
    B, S, D = x.shape
    N = B * S
    x2 = x.reshape(N, D)
    grid = (N // TILE,)
    wts, idx = pl.pallas_call(
        _router_body,
        grid=grid,
        in_specs=[
            pl.BlockSpec((TILE, D), lambda i: (i, 0)),
            pl.BlockSpec((D, E), lambda i: (0, 0)),
        ],
        out_specs=[
            pl.BlockSpec((TOP_K, TILE), lambda i: (0, i)),
            pl.BlockSpec((TOP_K, TILE), lambda i: (0, i)),
        ],
        out_shape=[
            jax.ShapeDtypeStruct((TOP_K, N), jnp.float32),
            jax.ShapeDtypeStruct((TOP_K, N), jnp.int32),
        ],
    )(x2, W_g)
    return wts.T.reshape(B, S, TOP_K), idx.T.reshape(B, S, TOP_K)


# speedup vs baseline: 1.0618x; 1.0618x over previous
import jax
import jax.numpy as jnp
from jax import lax
from jax.experimental import pallas as pl

E = 16
TOP_K = 2
CONF_THRESH = 0.7
TILE = 1024


def _router_body(x_ref, w_ref, wts_ref, idx_ref):
    logits = jnp.dot(x_ref[...], w_ref[...], preferred_element_type=jnp.float32)
    lt = logits.T
    m = jnp.max(lt, axis=0, keepdims=True)
    e = jnp.exp(lt - m)
    z = jnp.sum(e, axis=0, keepdims=True)
    probs = e / z

    iota = lax.broadcasted_iota(jnp.int32, probs.shape, 0)
    big = jnp.int32(E)
    w1 = jnp.max(probs, axis=0, keepdims=True)
    i1 = jnp.min(jnp.where(probs == w1, iota, big), axis=0, keepdims=True)
    masked = jnp.where(iota == i1, jnp.float32(-1.0), probs)
    w2 = jnp.max(masked, axis=0, keepdims=True)
    i2 = jnp.min(jnp.where(masked == w2, iota, big), axis=0, keepdims=True)

    keep = w1 >= CONF_THRESH
    i1 = jnp.where(keep, i1, -1)
    i2 = jnp.where(keep, i2, -1)

    wts_ref[...] = jnp.concatenate([w1, w2], axis=0)
    idx_ref[...] = jnp.concatenate([i1, i2], axis=0)


def kernel(x, W_g):
    B, S, D = x.shape
    N = B * S
    x2 = x.reshape(N, D)
    grid = (N // TILE,)
    wts, idx = pl.pallas_call(
        _router_body,
        grid=grid,
        in_specs=[
            pl.BlockSpec((TILE, D), lambda i: (i, 0)),
            pl.BlockSpec((D, E), lambda i: (0, 0)),
        ],
        out_specs=[
            pl.BlockSpec((TOP_K, TILE), lambda i: (0, i)),
            pl.BlockSpec((TOP_K, TILE), lambda i: (0, i)),
        ],
        out_shape=[
            jax.ShapeDtypeStruct((TOP_K, N), jnp.float32),
            jax.ShapeDtypeStruct((TOP_K, N), jnp.int32),
        ],
    )(x2, W_g)
    return wts.T.reshape(B, S, TOP_K), idx.T.reshape(B, S, TOP_K)
